# Initial kernel scaffold; baseline (speedup 1.0000x reference)
import jax
import jax.numpy as jnp
from jax.experimental import pallas as pl


def kernel(x, edges, W1, b1, W2, b2, Wl, bl):
    raise NotImplementedError("write your pallas kernel here")


# speedup vs baseline: 17.0480x; 17.0480x over previous
import functools

import jax
import jax.numpy as jnp
from jax import lax
from jax.experimental import pallas as pl
from jax.experimental.pallas import tpu as pltpu
from jax.experimental.pallas import tpu_sc as plsc

NC = 2
NS = 16
L = 16


def _deg_body(src_hbm, dst_hbm, o1_hbm, o2_hbm,
              h1, h2, srcv, dstv, red, res, sh1, sh2, sem, *, e_per_w, hsz):
    cid = lax.axis_index("c")
    sid = lax.axis_index("s")
    w = cid * NS + sid

    zero16 = jnp.zeros((L,), jnp.float32)

    @pl.loop(0, hsz // L)
    def _(i):
        h1[pl.ds(i * L, L)] = zero16
        h2[pl.ds(i * L, L)] = zero16

    ones = jnp.ones((L,), jnp.float32)
    base0 = w * e_per_w
    CH = 80

    @pl.loop(0, e_per_w // CH)
    def _(i):
        base = base0 + i * CH
        pltpu.sync_copy(src_hbm.at[pl.ds(base, CH)], srcv)
        pltpu.sync_copy(dst_hbm.at[pl.ds(base, CH)], dstv)
        for j in range(CH // L):
            s = srcv[pl.ds(j * L, L)]
            d = dstv[pl.ds(j * L, L)]
            plsc.addupdate_scatter(h1, [d], ones, mask=s < d)
            plsc.addupdate_scatter(h2, [d], ones, mask=d < s)

    pltpu.sync_copy(h1, sh1.at[sid])
    pltpu.sync_copy(h2, sh2.at[sid])
    plsc.subcore_barrier()

    chunk = hsz // NS
    c0 = sid * chunk
    for conv, (sh, o_hbm) in enumerate(((sh1, o1_hbm), (sh2, o2_hbm))):
        for k in range(NS):
            pltpu.sync_copy(sh.at[k, pl.ds(c0, chunk)], red.at[k])

        @pl.loop(0, chunk // L)
        def _(v):
            acc = red[0, pl.ds(v * L, L)]
            for k in range(1, NS):
                acc = acc + red[k, pl.ds(v * L, L)]
            res[pl.ds(v * L, L)] = acc

        pltpu.sync_copy(res, o_hbm.at[pl.ds(cid * hsz + c0, chunk)])


def _make_deg_kernel(E, hsz):
    e_per_w = E // (NC * NS)
    mesh = plsc.VectorSubcoreMesh(core_axis_name="c", subcore_axis_name="s")
    return pl.kernel(
        functools.partial(_deg_body, e_per_w=e_per_w, hsz=hsz),
        out_type=[jax.ShapeDtypeStruct((NC * hsz,), jnp.float32),
                  jax.ShapeDtypeStruct((NC * hsz,), jnp.float32)],
        mesh=mesh,
        compiler_params=pltpu.CompilerParams(needs_layout_passes=False),
        scratch_types=[
            pltpu.VMEM((hsz,), jnp.float32),
            pltpu.VMEM((hsz,), jnp.float32),
            pltpu.VMEM((80,), jnp.int32),
            pltpu.VMEM((80,), jnp.int32),
            pltpu.VMEM((NS, 1024), jnp.float32),
            pltpu.VMEM((1024,), jnp.float32),
            pltpu.VMEM_SHARED((NS, 16384), jnp.float32),
            pltpu.VMEM_SHARED((NS, 16384), jnp.float32),
            pltpu.SemaphoreType.DMA,
        ],
    )


def _edge_body(src_hbm, dst_hbm, z1_hbm, z2_hbm, u1_hbm, u2_hbm,
               acc, srcv, dstv, sidxv, rows, sem, *, n, e_per_s):
    cid = lax.axis_index("c")
    sid = lax.axis_index("s")
    cps = -(-n // NS) + 7 & ~7
    last = n - (NS - 1) * cps
    r0 = sid * cps

    def _init(z_hbm):
        @pl.when(sid < NS - 1)
        def _():
            pltpu.sync_copy(z_hbm.at[pl.ds(r0, cps)], acc.at[pl.ds(r0, cps)])

        @pl.when(sid == NS - 1)
        def _():
            pltpu.sync_copy(z_hbm.at[pl.ds((NS - 1) * cps, last)],
                            acc.at[pl.ds((NS - 1) * cps, last)])

    @pl.when(cid == 0)
    def _():
        _init(z1_hbm)

    @pl.when(cid == 1)
    def _():
        _init(z2_hbm)

    plsc.subcore_barrier()

    CH = 80
    base0 = sid * e_per_s
    trash = jnp.full((L,), n, jnp.int32)

    @pl.loop(0, e_per_s // CH)
    def _(i):
        base = base0 + i * CH
        pltpu.sync_copy(src_hbm.at[pl.ds(base, CH)], srcv)
        pltpu.sync_copy(dst_hbm.at[pl.ds(base, CH)], dstv)
        for j in range(CH // L):
            s = srcv[pl.ds(j * L, L)]
            d = dstv[pl.ds(j * L, L)]
            m = jnp.where(cid == 0, s < d, d < s)
            sidxv[pl.ds(j * L, L)] = jnp.where(m, d, trash)

        @pl.when(cid == 0)
        def _():
            pltpu.async_copy(z1_hbm.at[srcv], rows, sem).wait()

        @pl.when(cid == 1)
        def _():
            pltpu.async_copy(z2_hbm.at[srcv], rows, sem).wait()

        pltpu.sync_copy(rows, acc.at[sidxv], add=True)

    plsc.subcore_barrier()

    def _emit(u_hbm):
        @pl.when(sid < NS - 1)
        def _():
            pltpu.sync_copy(acc.at[pl.ds(r0, cps)], u_hbm.at[pl.ds(r0, cps)])

        @pl.when(sid == NS - 1)
        def _():
            pltpu.sync_copy(acc.at[pl.ds((NS - 1) * cps, last)],
                            u_hbm.at[pl.ds((NS - 1) * cps, last)])

    @pl.when(cid == 0)
    def _():
        _emit(u1_hbm)

    @pl.when(cid == 1)
    def _():
        _emit(u2_hbm)


def _make_edge_kernel(N, E, D):
    e_per_s = E // NS
    mesh = plsc.VectorSubcoreMesh(core_axis_name="c", subcore_axis_name="s")
    return pl.kernel(
        functools.partial(_edge_body, n=N, e_per_s=e_per_s),
        out_type=[jax.ShapeDtypeStruct((N, D), jnp.float32),
                  jax.ShapeDtypeStruct((N, D), jnp.float32)],
        mesh=mesh,
        compiler_params=pltpu.CompilerParams(needs_layout_passes=False),
        scratch_types=[
            pltpu.VMEM_SHARED((N + L, D), jnp.float32),
            pltpu.VMEM((80,), jnp.int32),
            pltpu.VMEM((80,), jnp.int32),
            pltpu.VMEM((80,), jnp.int32),
            pltpu.VMEM((80, D), jnp.float32),
            pltpu.SemaphoreType.DMA,
        ],
    )


def _zw_body(x_ref, w1_ref, w2_ref, dg1_ref, dg2_ref, z1_ref, z2_ref):
    xb = x_ref[...]
    dinv1 = lax.rsqrt(dg1_ref[...] + 1.0)
    dinv2 = lax.rsqrt(dg2_ref[...] + 1.0)
    z1_ref[...] = dinv1 * jnp.dot(xb, w1_ref[...], preferred_element_type=jnp.float32)
    z2_ref[...] = dinv2 * jnp.dot(xb, w2_ref[...], preferred_element_type=jnp.float32)


def _make_zw_kernel(N, D, BR):
    grid = (N // BR,)
    return pl.pallas_call(
        _zw_body,
        grid=grid,
        in_specs=[
            pl.BlockSpec((BR, D), lambda i: (i, 0)),
            pl.BlockSpec((D, D), lambda i: (0, 0)),
            pl.BlockSpec((D, D), lambda i: (0, 0)),
            pl.BlockSpec((BR, 1), lambda i: (i, 0)),
            pl.BlockSpec((BR, 1), lambda i: (i, 0)),
        ],
        out_specs=[
            pl.BlockSpec((BR, D), lambda i: (i, 0)),
            pl.BlockSpec((BR, D), lambda i: (i, 0)),
        ],
        out_shape=[jax.ShapeDtypeStruct((N, D), jnp.float32),
                   jax.ShapeDtypeStruct((N, D), jnp.float32)],
    )


def _fin_body(u1_ref, u2_ref, dg1_ref, dg2_ref, wl_ref, b1_ref, b2_ref, bl_ref,
              o_ref, *, D):
    dinv1 = lax.rsqrt(dg1_ref[...] + 1.0)
    dinv2 = lax.rsqrt(dg2_ref[...] + 1.0)
    wla = wl_ref[:D, :]
    wlb = wl_ref[D:, :]
    y = jnp.dot(dinv1 * u1_ref[...], wla, preferred_element_type=jnp.float32)
    y += jnp.dot(dinv2 * u2_ref[...], wlb, preferred_element_type=jnp.float32)
    cvec = (jnp.dot(b1_ref[...], wla, preferred_element_type=jnp.float32)
            + jnp.dot(b2_ref[...], wlb, preferred_element_type=jnp.float32)
            + bl_ref[...])
    o_ref[...] = y + cvec


def _make_fin_kernel(N, D, BR):
    grid = (N // BR,)
    return pl.pallas_call(
        functools.partial(_fin_body, D=D),
        grid=grid,
        in_specs=[
            pl.BlockSpec((BR, D), lambda i: (i, 0)),
            pl.BlockSpec((BR, D), lambda i: (i, 0)),
            pl.BlockSpec((BR, 1), lambda i: (i, 0)),
            pl.BlockSpec((BR, 1), lambda i: (i, 0)),
            pl.BlockSpec((2 * D, D), lambda i: (0, 0)),
            pl.BlockSpec((1, D), lambda i: (0, 0)),
            pl.BlockSpec((1, D), lambda i: (0, 0)),
            pl.BlockSpec((1, D), lambda i: (0, 0)),
        ],
        out_specs=pl.BlockSpec((BR, D), lambda i: (i, 0)),
        out_shape=jax.ShapeDtypeStruct((N, D), jnp.float32),
    )


def kernel(x, edges, W1, b1, W2, b2, Wl, bl):
    N, D = x.shape
    E = edges.shape[1]
    hsz = 16384
    assert N <= hsz and E % (NC * NS * 80) == 0 and N % NS == 0

    src = edges[0]
    dst = edges[1]

    o1, o2 = _make_deg_kernel(E, hsz)(src, dst)
    deg1 = (o1[:hsz] + o1[hsz:])[:N].reshape(N, 1)
    deg2 = (o2[:hsz] + o2[hsz:])[:N].reshape(N, 1)

    z1, z2 = _make_zw_kernel(N, D, 1000)(x, W1, W2, deg1, deg2)
    u1, u2 = _make_edge_kernel(N, E, D)(src, dst, z1, z2)
    return _make_fin_kernel(N, D, 1000)(
        u1, u2, deg1, deg2, Wl,
        b1.reshape(1, D), b2.reshape(1, D), bl.reshape(1, D))


# speedup vs baseline: 41.7755x; 2.4505x over previous
import functools

import jax
import jax.numpy as jnp
from jax import lax
from jax.experimental import pallas as pl
from jax.experimental.pallas import tpu as pltpu
from jax.experimental.pallas import tpu_sc as plsc

NC = 2
NS = 16
L = 16


def _deg_body(src_hbm, dst_hbm, o1_hbm, o2_hbm, cs1_hbm, cd1_hbm, cs2_hbm,
              cd2_hbm, cnt_hbm,
              h1, h2, se, de, bs1, bd1, bs2, bd2, cntv, red, res, sh1, sh2,
              sem, *, e_per_w, hsz, n, rstride, pair):
    cid = lax.axis_index("c")
    sid = lax.axis_index("s")
    w = cid * NS + sid

    zero16 = jnp.zeros((L,), jnp.float32)

    @pl.loop(0, hsz // L)
    def _(i):
        h1[pl.ds(i * L, L)] = zero16
        h2[pl.ds(i * L, L)] = zero16

    pltpu.sync_copy(src_hbm.at[pl.ds(w * e_per_w, e_per_w)], se)
    pltpu.sync_copy(dst_hbm.at[pl.ds(w * e_per_w, e_per_w)], de)

    ones = jnp.ones((L,), jnp.float32)

    def step(i, carry):
        c1, c2 = carry
        s = se[pl.ds(i * L, L)]
        d = de[pl.ds(i * L, L)]
        m1 = s < d
        m2 = d < s
        plsc.addupdate_scatter(h1, [d], ones, mask=m1)
        plsc.addupdate_scatter(h2, [d], ones, mask=m2)
        plsc.store_compressed(bs1.at[pl.ds(c1, L)], s, mask=m1)
        plsc.store_compressed(bd1.at[pl.ds(c1, L)], d, mask=m1)
        plsc.store_compressed(bs2.at[pl.ds(c2, L)], s, mask=m2)
        plsc.store_compressed(bd2.at[pl.ds(c2, L)], d, mask=m2)
        c1 = c1 + jnp.sum(m1.astype(jnp.int32))
        c2 = c2 + jnp.sum(m2.astype(jnp.int32))
        return c1, c2

    c1, c2 = pl.loop(0, e_per_w // L,
                     init_carry=(jnp.int32(0), jnp.int32(0)))(step)

    sent_d = jnp.full((L,), n, jnp.int32)
    zs = jnp.zeros((L,), jnp.int32)
    for k in range(pair // L):
        bs1[pl.ds(c1 + k * L, L)] = zs
        bd1[pl.ds(c1 + k * L, L)] = sent_d
        bs2[pl.ds(c2 + k * L, L)] = zs
        bd2[pl.ds(c2 + k * L, L)] = sent_d
    p1 = ((c1 + pair - 1) // pair) * pair
    p2 = ((c2 + pair - 1) // pair) * pair
    lane = jnp.arange(L, dtype=jnp.int32)
    cntv[...] = jnp.where(lane == 0, p1, jnp.where(lane == 1, p2, 0))

    pltpu.sync_copy(cntv, cnt_hbm.at[pl.ds(w * L, L)])
    pltpu.sync_copy(bs1, cs1_hbm.at[pl.ds(w * rstride, rstride)])
    pltpu.sync_copy(bd1, cd1_hbm.at[pl.ds(w * rstride, rstride)])
    pltpu.sync_copy(bs2, cs2_hbm.at[pl.ds(w * rstride, rstride)])
    pltpu.sync_copy(bd2, cd2_hbm.at[pl.ds(w * rstride, rstride)])

    pltpu.sync_copy(h1, sh1.at[sid])
    pltpu.sync_copy(h2, sh2.at[sid])
    plsc.subcore_barrier()

    chunk = hsz // NS
    c0 = sid * chunk
    for conv, (sh, o_hbm) in enumerate(((sh1, o1_hbm), (sh2, o2_hbm))):
        for k in range(NS):
            pltpu.sync_copy(sh.at[k, pl.ds(c0, chunk)], red.at[k])

        @pl.loop(0, chunk // L)
        def _(v):
            acc = red[0, pl.ds(v * L, L)]
            for k in range(1, NS):
                acc = acc + red[k, pl.ds(v * L, L)]
            res[pl.ds(v * L, L)] = acc

        pltpu.sync_copy(res, o_hbm.at[pl.ds(cid * hsz + c0, chunk)])


def _make_deg_kernel(E, hsz, n, rstride, pair):
    e_per_w = E // (NC * NS)
    NW = NC * NS
    mesh = plsc.VectorSubcoreMesh(core_axis_name="c", subcore_axis_name="s")
    return pl.kernel(
        functools.partial(_deg_body, e_per_w=e_per_w, hsz=hsz, n=n,
                          rstride=rstride, pair=pair),
        out_type=[jax.ShapeDtypeStruct((NC * hsz,), jnp.float32),
                  jax.ShapeDtypeStruct((NC * hsz,), jnp.float32),
                  jax.ShapeDtypeStruct((NW * rstride,), jnp.int32),
                  jax.ShapeDtypeStruct((NW * rstride,), jnp.int32),
                  jax.ShapeDtypeStruct((NW * rstride,), jnp.int32),
                  jax.ShapeDtypeStruct((NW * rstride,), jnp.int32),
                  jax.ShapeDtypeStruct((NW * L,), jnp.int32)],
        mesh=mesh,
        compiler_params=pltpu.CompilerParams(needs_layout_passes=False),
        scratch_types=[
            pltpu.VMEM((hsz,), jnp.float32),
            pltpu.VMEM((hsz,), jnp.float32),
            pltpu.VMEM((e_per_w,), jnp.int32),
            pltpu.VMEM((e_per_w,), jnp.int32),
            pltpu.VMEM((rstride,), jnp.int32),
            pltpu.VMEM((rstride,), jnp.int32),
            pltpu.VMEM((rstride,), jnp.int32),
            pltpu.VMEM((rstride,), jnp.int32),
            pltpu.VMEM((L,), jnp.int32),
            pltpu.VMEM((NS, hsz // NS), jnp.float32),
            pltpu.VMEM((hsz // NS,), jnp.float32),
            pltpu.VMEM_SHARED((NS, hsz), jnp.float32),
            pltpu.VMEM_SHARED((NS, hsz), jnp.float32),
            pltpu.SemaphoreType.DMA,
        ],
    )


def _edge_body(z1_hbm, z2_hbm, cs1_hbm, cd1_hbm, cs2_hbm, cd2_hbm, cnt_hbm,
               u1_hbm, u2_hbm,
               acc, ebs, ebd, cnts, gx0, sx0, gx1, sx1, rb0, rb1,
               sem0, sem1, *, n, rstride, ch):
    cid = lax.axis_index("c")
    sid = lax.axis_index("s")
    cps = -(-n // NS) + 7 & ~7
    last = n - (NS - 1) * cps
    r0 = sid * cps

    def _init(z_hbm):
        @pl.when(sid < NS - 1)
        def _():
            pltpu.sync_copy(z_hbm.at[pl.ds(r0, cps)], acc.at[pl.ds(r0, cps)])

        @pl.when(sid == NS - 1)
        def _():
            pltpu.sync_copy(z_hbm.at[pl.ds((NS - 1) * cps, last)],
                            acc.at[pl.ds((NS - 1) * cps, last)])

    @pl.when(cid == 0)
    def _():
        _init(z1_hbm)

    @pl.when(cid == 1)
    def _():
        _init(z2_hbm)

    pltpu.sync_copy(cnt_hbm, cnts)
    plsc.subcore_barrier()

    def cp_idx(i, gx, sx):
        for j in range(ch // L):
            gx[pl.ds(j * L, L)] = ebs[pl.ds(i * ch + j * L, L)]
            sx[pl.ds(j * L, L)] = ebd[pl.ds(i * ch + j * L, L)]

    def gstart(gx, rbuf, sem):
        @pl.when(cid == 0)
        def _():
            pltpu.async_copy(z1_hbm.at[gx], rbuf, sem)

        @pl.when(cid == 1)
        def _():
            pltpu.async_copy(z2_hbm.at[gx], rbuf, sem)

    def gwait(rbuf, sem):
        pltpu.make_async_copy(z1_hbm.at[pl.ds(0, ch)], rbuf, sem).wait()

    def scat(rbuf, sx):
        pltpu.sync_copy(rbuf, acc.at[sx], add=True)

    for rr in range(2):
        w = sid * 2 + rr

        @pl.when(cid == 0)
        def _():
            pltpu.sync_copy(cs1_hbm.at[pl.ds(w * rstride, rstride)], ebs)
            pltpu.sync_copy(cd1_hbm.at[pl.ds(w * rstride, rstride)], ebd)

        @pl.when(cid == 1)
        def _():
            pltpu.sync_copy(cs2_hbm.at[pl.ds(w * rstride, rstride)], ebs)
            pltpu.sync_copy(cd2_hbm.at[pl.ds(w * rstride, rstride)], ebd)

        cv = cnts[pl.ds(w * L, L)]
        nch = jnp.where(cid == 0, cv[0], cv[1]) // ch

        @pl.when(nch > 0)
        def _():
            cp_idx(0, gx0, sx0)
            gstart(gx0, rb0, sem0)

        @pl.loop(0, nch // 2)
        def _(g):
            i0 = g * 2
            cp_idx(i0 + 1, gx1, sx1)
            gstart(gx1, rb1, sem1)
            gwait(rb0, sem0)
            scat(rb0, sx0)

            @pl.when(i0 + 2 < nch)
            def _():
                cp_idx(i0 + 2, gx0, sx0)
                gstart(gx0, rb0, sem0)

            gwait(rb1, sem1)
            scat(rb1, sx1)

    plsc.subcore_barrier()

    def _emit(u_hbm):
        @pl.when(sid < NS - 1)
        def _():
            pltpu.sync_copy(acc.at[pl.ds(r0, cps)], u_hbm.at[pl.ds(r0, cps)])

        @pl.when(sid == NS - 1)
        def _():
            pltpu.sync_copy(acc.at[pl.ds((NS - 1) * cps, last)],
                            u_hbm.at[pl.ds((NS - 1) * cps, last)])

    @pl.when(cid == 0)
    def _():
        _emit(u1_hbm)

    @pl.when(cid == 1)
    def _():
        _emit(u2_hbm)


def _make_edge_kernel(N, D, rstride, ch):
    NW = NC * NS
    mesh = plsc.VectorSubcoreMesh(core_axis_name="c", subcore_axis_name="s")
    return pl.kernel(
        functools.partial(_edge_body, n=N, rstride=rstride, ch=ch),
        out_type=[jax.ShapeDtypeStruct((N, D), jnp.float32),
                  jax.ShapeDtypeStruct((N, D), jnp.float32)],
        mesh=mesh,
        compiler_params=pltpu.CompilerParams(needs_layout_passes=False),
        scratch_types=[
            pltpu.VMEM_SHARED((N + L, D), jnp.float32),
            pltpu.VMEM((rstride,), jnp.int32),
            pltpu.VMEM((rstride,), jnp.int32),
            pltpu.VMEM((NW * L,), jnp.int32),
            pltpu.VMEM((ch,), jnp.int32),
            pltpu.VMEM((ch,), jnp.int32),
            pltpu.VMEM((ch,), jnp.int32),
            pltpu.VMEM((ch,), jnp.int32),
            pltpu.VMEM((ch, D), jnp.float32),
            pltpu.VMEM((ch, D), jnp.float32),
            pltpu.SemaphoreType.DMA,
            pltpu.SemaphoreType.DMA,
        ],
    )


def _zw_body(x_ref, w1_ref, w2_ref, dg1_ref, dg2_ref, z1_ref, z2_ref):
    xb = x_ref[...]
    dinv1 = lax.rsqrt(dg1_ref[...] + 1.0)
    dinv2 = lax.rsqrt(dg2_ref[...] + 1.0)
    z1_ref[...] = dinv1 * jnp.dot(xb, w1_ref[...],
                                  preferred_element_type=jnp.float32)
    z2_ref[...] = dinv2 * jnp.dot(xb, w2_ref[...],
                                  preferred_element_type=jnp.float32)


def _make_zw_kernel(N, D, BR):
    grid = (N // BR,)
    return pl.pallas_call(
        _zw_body,
        grid=grid,
        in_specs=[
            pl.BlockSpec((BR, D), lambda i: (i, 0)),
            pl.BlockSpec((D, D), lambda i: (0, 0)),
            pl.BlockSpec((D, D), lambda i: (0, 0)),
            pl.BlockSpec((BR, 1), lambda i: (i, 0)),
            pl.BlockSpec((BR, 1), lambda i: (i, 0)),
        ],
        out_specs=[
            pl.BlockSpec((BR, D), lambda i: (i, 0)),
            pl.BlockSpec((BR, D), lambda i: (i, 0)),
        ],
        out_shape=[jax.ShapeDtypeStruct((N, D), jnp.float32),
                   jax.ShapeDtypeStruct((N, D), jnp.float32)],
    )


def _fin_body(u1_ref, u2_ref, dg1_ref, dg2_ref, wl_ref, b1_ref, b2_ref,
              bl_ref, o_ref, *, D):
    dinv1 = lax.rsqrt(dg1_ref[...] + 1.0)
    dinv2 = lax.rsqrt(dg2_ref[...] + 1.0)
    wla = wl_ref[:D, :]
    wlb = wl_ref[D:, :]
    y = jnp.dot(dinv1 * u1_ref[...], wla, preferred_element_type=jnp.float32)
    y += jnp.dot(dinv2 * u2_ref[...], wlb, preferred_element_type=jnp.float32)
    cvec = (jnp.dot(b1_ref[...], wla, preferred_element_type=jnp.float32)
            + jnp.dot(b2_ref[...], wlb, preferred_element_type=jnp.float32)
            + bl_ref[...])
    o_ref[...] = y + cvec


def _make_fin_kernel(N, D, BR):
    grid = (N // BR,)
    return pl.pallas_call(
        functools.partial(_fin_body, D=D),
        grid=grid,
        in_specs=[
            pl.BlockSpec((BR, D), lambda i: (i, 0)),
            pl.BlockSpec((BR, D), lambda i: (i, 0)),
            pl.BlockSpec((BR, 1), lambda i: (i, 0)),
            pl.BlockSpec((BR, 1), lambda i: (i, 0)),
            pl.BlockSpec((2 * D, D), lambda i: (0, 0)),
            pl.BlockSpec((1, D), lambda i: (0, 0)),
            pl.BlockSpec((1, D), lambda i: (0, 0)),
            pl.BlockSpec((1, D), lambda i: (0, 0)),
        ],
        out_specs=pl.BlockSpec((BR, D), lambda i: (i, 0)),
        out_shape=jax.ShapeDtypeStruct((N, D), jnp.float32),
    )


def kernel(x, edges, W1, b1, W2, b2, Wl, bl):
    N, D = x.shape
    E = edges.shape[1]
    hsz = 10240
    ch = 96
    pair = 2 * ch
    e_per_w = E // (NC * NS)
    rstride = e_per_w + pair
    assert N <= hsz and E % (NC * NS * L) == 0 and N % NS == 0
    assert rstride % 8 == 0

    src = edges[0]
    dst = edges[1]

    o1, o2, cs1, cd1, cs2, cd2, cnt = _make_deg_kernel(
        E, hsz, N, rstride, pair)(src, dst)
    deg1 = (o1[:hsz] + o1[hsz:])[:N].reshape(N, 1)
    deg2 = (o2[:hsz] + o2[hsz:])[:N].reshape(N, 1)

    z1, z2 = _make_zw_kernel(N, D, 1000)(x, W1, W2, deg1, deg2)
    u1, u2 = _make_edge_kernel(N, D, rstride, ch)(z1, z2, cs1, cd1, cs2,
                                                  cd2, cnt)
    return _make_fin_kernel(N, D, 1000)(
        u1, u2, deg1, deg2, Wl,
        b1.reshape(1, D), b2.reshape(1, D), bl.reshape(1, D))
